# baseline (speedup 1.0000x reference)
import functools

import jax
import jax.numpy as jnp
from jax import lax
from jax.experimental import pallas as pl
from jax.experimental.pallas import tpu as pltpu
from jax.experimental.pallas import tpu_sc as plsc

NC = 2
NS = 16
NW = NC * NS
L = 16

HB1 = 32768
HB2 = 65536
CH = 8192


def _phase_a(x_hbm, y_hbm, zeros_hbm, keys_hbm, hist_hbm, x_v, y_v, k_v,
             hist_v, *, per_w):
    c = lax.axis_index("c")
    s = lax.axis_index("s")
    wid = s * NC + c
    pltpu.sync_copy(zeros_hbm.at[pl.ds(0, HB1)], hist_v)
    ones = jnp.ones((L,), jnp.int32)

    for ch in range(per_w // CH):
        base = wid * per_w + ch * CH
        pltpu.sync_copy(x_hbm.at[pl.ds(base, CH)], x_v)
        pltpu.sync_copy(y_hbm.at[pl.ds(base, CH)], y_v)

        def body(i, _):
            sl = pl.ds(i * L, L)
            d = x_v[sl] - y_v[sl]
            a = jnp.abs(d)
            loss = jnp.where(a < 1.0, 0.5 * d * d, a - 0.5)
            key = plsc.bitcast(loss, jnp.int32)
            k_v[sl] = key
            idx = lax.shift_right_logical(key, 16)
            plsc.addupdate_scatter(hist_v, [idx], ones)
            return 0

        lax.fori_loop(0, CH // L, body, 0)
        pltpu.sync_copy(k_v, keys_hbm.at[pl.ds(base, CH)])

    pltpu.sync_copy(hist_v, hist_hbm.at[wid])


def _phase_b(hist_ref, out_ref, *, k):
    h = jnp.sum(hist_ref[...], axis=0)
    rows, cols = h.shape
    idx = (lax.broadcasted_iota(jnp.int32, h.shape, 0) * cols
           + lax.broadcasted_iota(jnp.int32, h.shape, 1))

    def step(i, prefix):
        trial = prefix + (jnp.int32(1) << (jnp.int32(14) - i))
        cnt = jnp.sum(jnp.where(idx >= trial, h, 0))
        return jnp.where(cnt >= k, trial, prefix)

    bstar = lax.fori_loop(0, 15, step, jnp.int32(0))
    n_gt1 = jnp.sum(jnp.where(idx > bstar, h, 0))
    rowid = lax.broadcasted_iota(jnp.int32, (8, 128), 0)
    out_ref[...] = jnp.where(rowid == 0, bstar, n_gt1)


def _phase_c(keys_hbm, sel_hbm, zeros_hbm, hist2_hbm, sabove_hbm, k_v, sel_v,
             hist2_v, *, per_w):
    c = lax.axis_index("c")
    s = lax.axis_index("s")
    wid = s * NC + c
    pltpu.sync_copy(zeros_hbm.at[pl.ds(0, HB2)], hist2_v)
    pltpu.sync_copy(sel_hbm.at[pl.ds(0, L)], sel_v)
    bs_vec = sel_v[...]
    ones = jnp.ones((L,), jnp.int32)

    acc = jnp.zeros((L,), jnp.float32)
    for ch in range(per_w // CH):
        base = wid * per_w + ch * CH
        pltpu.sync_copy(keys_hbm.at[pl.ds(base, CH)], k_v)

        def body(i, a):
            key = k_v[pl.ds(i * L, L)]
            hi = lax.shift_right_logical(key, 16)
            loss = plsc.bitcast(key, jnp.float32)
            a = a + jnp.where(hi > bs_vec, loss, jnp.float32(0.0))
            low = jnp.bitwise_and(key, jnp.int32(0xFFFF))
            plsc.addupdate_scatter(hist2_v, [low], ones, mask=hi == bs_vec)
            return a

        acc = lax.fori_loop(0, CH // L, body, acc)

    k_v[pl.ds(0, L)] = plsc.bitcast(acc, jnp.int32)
    pltpu.sync_copy(k_v.at[pl.ds(0, L)],
                    sabove_hbm.at[pl.ds(wid * L, L)])
    pltpu.sync_copy(hist2_v, hist2_hbm.at[wid])


def _phase_d(hist2_ref, sel_ref, sab_ref, out_ref, *, k):
    h2 = jnp.sum(hist2_ref[...], axis=0)
    rows, cols = h2.shape
    j = (lax.broadcasted_iota(jnp.int32, h2.shape, 0) * cols
         + lax.broadcasted_iota(jnp.int32, h2.shape, 1))
    bstar = sel_ref[0, 0]
    n_gt1 = sel_ref[1, 0]
    s_above = jnp.sum(lax.bitcast_convert_type(sab_ref[...], jnp.float32))
    r1 = k - n_gt1

    def step(i, prefix):
        trial = prefix + (jnp.int32(1) << (jnp.int32(15) - i))
        cnt = jnp.sum(jnp.where(j >= trial, h2, 0))
        return jnp.where(cnt >= r1, trial, prefix)

    low = lax.fori_loop(0, 16, step, jnp.int32(0))
    vals = lax.bitcast_convert_type((bstar << 16) + j, jnp.float32)
    above = j > low
    n2 = jnp.sum(jnp.where(above, h2, 0))
    s2 = jnp.sum(jnp.where(above, h2.astype(jnp.float32) * vals, 0.0))
    t_val = lax.bitcast_convert_type((bstar << 16) + low, jnp.float32)
    rem = (r1 - n2).astype(jnp.float32)
    out_ref[0, 0] = (s_above + s2 + rem * t_val) / jnp.float32(k)


def kernel(inputs, targets):
    n_total = inputs.size
    k = int(0.6 * n_total)
    per_w = n_total // NW
    x = inputs.reshape(n_total)
    y = targets.reshape(n_total)
    zeros = jnp.zeros((HB2,), jnp.int32)

    mesh = plsc.VectorSubcoreMesh(core_axis_name="c", subcore_axis_name="s")
    sc_params = pltpu.CompilerParams(needs_layout_passes=False)

    keys, hist1 = pl.kernel(
        functools.partial(_phase_a, per_w=per_w),
        mesh=mesh,
        compiler_params=sc_params,
        out_type=[jax.ShapeDtypeStruct((n_total,), jnp.int32),
                  jax.ShapeDtypeStruct((NW, HB1), jnp.int32)],
        scratch_types=[pltpu.VMEM((CH,), jnp.float32),
                       pltpu.VMEM((CH,), jnp.float32),
                       pltpu.VMEM((CH,), jnp.int32),
                       pltpu.VMEM((HB1,), jnp.int32)],
    )(x, y, zeros)

    sel = pl.pallas_call(
        functools.partial(_phase_b, k=k),
        out_shape=jax.ShapeDtypeStruct((8, 128), jnp.int32),
        in_specs=[pl.BlockSpec(memory_space=pltpu.VMEM)],
        out_specs=pl.BlockSpec(memory_space=pltpu.VMEM),
    )(hist1.reshape(NW, HB1 // 128, 128))

    hist2, sab = pl.kernel(
        functools.partial(_phase_c, per_w=per_w),
        mesh=mesh,
        compiler_params=sc_params,
        out_type=[jax.ShapeDtypeStruct((NW, HB2), jnp.int32),
                  jax.ShapeDtypeStruct((NW * L,), jnp.int32)],
        scratch_types=[pltpu.VMEM((CH,), jnp.int32),
                       pltpu.VMEM((L,), jnp.int32),
                       pltpu.VMEM((HB2,), jnp.int32)],
    )(keys, sel.reshape(HB1 // 32), zeros)

    out = pl.pallas_call(
        functools.partial(_phase_d, k=k),
        out_shape=jax.ShapeDtypeStruct((1, 1), jnp.float32),
        in_specs=[pl.BlockSpec(memory_space=pltpu.VMEM),
                  pl.BlockSpec(memory_space=pltpu.VMEM),
                  pl.BlockSpec(memory_space=pltpu.VMEM)],
        out_specs=pl.BlockSpec(memory_space=pltpu.SMEM),
    )(hist2.reshape(NW, HB2 // 128, 128), sel,
      sab.reshape(NW * L // 128, 128))

    return out[0, 0]


# speedup vs baseline: 1.0139x; 1.0139x over previous
import functools

import jax
import jax.numpy as jnp
from jax import lax
from jax.experimental import pallas as pl
from jax.experimental.pallas import tpu as pltpu
from jax.experimental.pallas import tpu_sc as plsc

NC = 2
NS = 16
NW = NC * NS
L = 16

HB1 = 32768
HB2 = 65536
CH = 8192


def _phase_a(x_hbm, y_hbm, zeros_hbm, keys_hbm, hist_hbm, x_v, y_v, k_v,
             hist_v, *, per_w):
    c = lax.axis_index("c")
    s = lax.axis_index("s")
    wid = s * NC + c
    pltpu.sync_copy(zeros_hbm.at[pl.ds(0, HB1)], hist_v)
    ones = jnp.ones((L,), jnp.int32)

    for ch in range(per_w // CH):
        base = wid * per_w + ch * CH
        pltpu.sync_copy(x_hbm.at[pl.ds(base, CH)], x_v)
        pltpu.sync_copy(y_hbm.at[pl.ds(base, CH)], y_v)

        def body(i, _):
            sl = pl.ds(i * L, L)
            d = x_v[sl] - y_v[sl]
            a = jnp.abs(d)
            loss = jnp.where(a < 1.0, 0.5 * d * d, a - 0.5)
            key = plsc.bitcast(loss, jnp.int32)
            k_v[sl] = key
            idx = lax.shift_right_logical(key, 16)
            plsc.addupdate_scatter(hist_v, [idx], ones)
            return 0

        lax.fori_loop(0, CH // L, body, 0, unroll=8)
        pltpu.sync_copy(k_v, keys_hbm.at[pl.ds(base, CH)])

    pltpu.sync_copy(hist_v, hist_hbm.at[wid])


def _phase_b(hist_ref, out_ref, *, k):
    h = jnp.sum(hist_ref[...], axis=0)
    rows, cols = h.shape
    idx = (lax.broadcasted_iota(jnp.int32, h.shape, 0) * cols
           + lax.broadcasted_iota(jnp.int32, h.shape, 1))

    def step(i, prefix):
        trial = prefix + (jnp.int32(1) << (jnp.int32(14) - i))
        cnt = jnp.sum(jnp.where(idx >= trial, h, 0))
        return jnp.where(cnt >= k, trial, prefix)

    bstar = lax.fori_loop(0, 15, step, jnp.int32(0))
    n_gt1 = jnp.sum(jnp.where(idx > bstar, h, 0))
    rowid = lax.broadcasted_iota(jnp.int32, (8, 128), 0)
    out_ref[...] = jnp.where(rowid == 0, bstar, n_gt1)


def _phase_c(keys_hbm, sel_hbm, zeros_hbm, hist2_hbm, sabove_hbm, k_v, sel_v,
             hist2_v, *, per_w):
    c = lax.axis_index("c")
    s = lax.axis_index("s")
    wid = s * NC + c
    pltpu.sync_copy(zeros_hbm.at[pl.ds(0, HB2)], hist2_v)
    pltpu.sync_copy(sel_hbm.at[pl.ds(0, L)], sel_v)
    bs_vec = sel_v[...]
    ones = jnp.ones((L,), jnp.int32)

    acc = jnp.zeros((L,), jnp.float32)
    for ch in range(per_w // CH):
        base = wid * per_w + ch * CH
        pltpu.sync_copy(keys_hbm.at[pl.ds(base, CH)], k_v)

        def body(i, a):
            key = k_v[pl.ds(i * L, L)]
            hi = lax.shift_right_logical(key, 16)
            loss = plsc.bitcast(key, jnp.float32)
            a = a + jnp.where(hi > bs_vec, loss, jnp.float32(0.0))
            low = jnp.bitwise_and(key, jnp.int32(0xFFFF))
            plsc.addupdate_scatter(hist2_v, [low], ones, mask=hi == bs_vec)
            return a

        acc = lax.fori_loop(0, CH // L, body, acc, unroll=8)

    k_v[pl.ds(0, L)] = plsc.bitcast(acc, jnp.int32)
    pltpu.sync_copy(k_v.at[pl.ds(0, L)],
                    sabove_hbm.at[pl.ds(wid * L, L)])
    pltpu.sync_copy(hist2_v, hist2_hbm.at[wid])


def _phase_d(hist2_ref, sel_ref, sab_ref, out_ref, *, k):
    h2 = jnp.sum(hist2_ref[...], axis=0)
    rows, cols = h2.shape
    j = (lax.broadcasted_iota(jnp.int32, h2.shape, 0) * cols
         + lax.broadcasted_iota(jnp.int32, h2.shape, 1))
    bstar = sel_ref[0, 0]
    n_gt1 = sel_ref[1, 0]
    s_above = jnp.sum(lax.bitcast_convert_type(sab_ref[...], jnp.float32))
    r1 = k - n_gt1

    def step(i, prefix):
        trial = prefix + (jnp.int32(1) << (jnp.int32(15) - i))
        cnt = jnp.sum(jnp.where(j >= trial, h2, 0))
        return jnp.where(cnt >= r1, trial, prefix)

    low = lax.fori_loop(0, 16, step, jnp.int32(0))
    vals = lax.bitcast_convert_type((bstar << 16) + j, jnp.float32)
    above = j > low
    n2 = jnp.sum(jnp.where(above, h2, 0))
    s2 = jnp.sum(jnp.where(above, h2.astype(jnp.float32) * vals, 0.0))
    t_val = lax.bitcast_convert_type((bstar << 16) + low, jnp.float32)
    rem = (r1 - n2).astype(jnp.float32)
    out_ref[0, 0] = (s_above + s2 + rem * t_val) / jnp.float32(k)


def kernel(inputs, targets):
    n_total = inputs.size
    k = int(0.6 * n_total)
    per_w = n_total // NW
    x = inputs.reshape(n_total)
    y = targets.reshape(n_total)
    zeros = jnp.zeros((HB2,), jnp.int32)

    mesh = plsc.VectorSubcoreMesh(core_axis_name="c", subcore_axis_name="s")
    sc_params = pltpu.CompilerParams(needs_layout_passes=False)

    keys, hist1 = pl.kernel(
        functools.partial(_phase_a, per_w=per_w),
        mesh=mesh,
        compiler_params=sc_params,
        out_type=[jax.ShapeDtypeStruct((n_total,), jnp.int32),
                  jax.ShapeDtypeStruct((NW, HB1), jnp.int32)],
        scratch_types=[pltpu.VMEM((CH,), jnp.float32),
                       pltpu.VMEM((CH,), jnp.float32),
                       pltpu.VMEM((CH,), jnp.int32),
                       pltpu.VMEM((HB1,), jnp.int32)],
    )(x, y, zeros)

    sel = pl.pallas_call(
        functools.partial(_phase_b, k=k),
        out_shape=jax.ShapeDtypeStruct((8, 128), jnp.int32),
        in_specs=[pl.BlockSpec(memory_space=pltpu.VMEM)],
        out_specs=pl.BlockSpec(memory_space=pltpu.VMEM),
    )(hist1.reshape(NW, HB1 // 128, 128))

    hist2, sab = pl.kernel(
        functools.partial(_phase_c, per_w=per_w),
        mesh=mesh,
        compiler_params=sc_params,
        out_type=[jax.ShapeDtypeStruct((NW, HB2), jnp.int32),
                  jax.ShapeDtypeStruct((NW * L,), jnp.int32)],
        scratch_types=[pltpu.VMEM((CH,), jnp.int32),
                       pltpu.VMEM((L,), jnp.int32),
                       pltpu.VMEM((HB2,), jnp.int32)],
    )(keys, sel.reshape(HB1 // 32), zeros)

    out = pl.pallas_call(
        functools.partial(_phase_d, k=k),
        out_shape=jax.ShapeDtypeStruct((1, 1), jnp.float32),
        in_specs=[pl.BlockSpec(memory_space=pltpu.VMEM),
                  pl.BlockSpec(memory_space=pltpu.VMEM),
                  pl.BlockSpec(memory_space=pltpu.VMEM)],
        out_specs=pl.BlockSpec(memory_space=pltpu.SMEM),
    )(hist2.reshape(NW, HB2 // 128, 128), sel,
      sab.reshape(NW * L // 128, 128))

    return out[0, 0]


# speedup vs baseline: 1.3512x; 1.3326x over previous
import functools

import jax
import jax.numpy as jnp
from jax import lax
from jax.experimental import pallas as pl
from jax.experimental.pallas import tpu as pltpu
from jax.experimental.pallas import tpu_sc as plsc

NC = 2
NS = 16
NW = NC * NS
L = 16

HB1 = 32768
HB2 = 65536
CH = 8192


def _phase_a(x_hbm, y_hbm, zeros_hbm, zerosf_hbm, keys_hbm, hist_hbm,
             shist_hbm, x_v, y_v, k_v, hist_v, shist_v, *, per_w):
    c = lax.axis_index("c")
    s = lax.axis_index("s")
    wid = s * NC + c
    pltpu.sync_copy(zeros_hbm.at[pl.ds(0, HB1)], hist_v)
    pltpu.sync_copy(zerosf_hbm.at[pl.ds(0, HB1)], shist_v)
    ones = jnp.ones((L,), jnp.int32)

    for ch in range(per_w // CH):
        base = wid * per_w + ch * CH
        pltpu.sync_copy(x_hbm.at[pl.ds(base, CH)], x_v)
        pltpu.sync_copy(y_hbm.at[pl.ds(base, CH)], y_v)

        def body(i):
            sl = pl.ds(i * L, L)
            d = x_v[sl] - y_v[sl]
            a = jnp.abs(d)
            loss = jnp.where(a < 1.0, 0.5 * d * d, a - 0.5)
            key = plsc.bitcast(loss, jnp.int32)
            k_v[sl] = key
            idx = lax.shift_right_logical(key, 16)
            plsc.addupdate_scatter(hist_v, [idx], ones)
            plsc.addupdate_scatter(shist_v, [idx], loss)

        plsc.parallel_loop(0, CH // L, 1, unroll=8)(body)
        pltpu.sync_copy(k_v, keys_hbm.at[pl.ds(base, CH)])

    pltpu.sync_copy(hist_v, hist_hbm.at[wid])
    pltpu.sync_copy(shist_v, shist_hbm.at[wid])


def _phase_b(hist_ref, shist_ref, out_ref, *, k):
    h = jnp.sum(hist_ref[...], axis=0)
    sh = jnp.sum(shist_ref[...], axis=0)
    idx = (lax.broadcasted_iota(jnp.int32, h.shape, 0) * h.shape[1]
           + lax.broadcasted_iota(jnp.int32, h.shape, 1))

    def step(i, prefix):
        trial = prefix + (jnp.int32(1) << (jnp.int32(14) - i))
        cnt = jnp.sum(jnp.where(idx >= trial, h, 0))
        return jnp.where(cnt >= k, trial, prefix)

    bstar = lax.fori_loop(0, 15, step, jnp.int32(0))
    above = idx > bstar
    n_gt1 = jnp.sum(jnp.where(above, h, 0))
    s_above = jnp.sum(jnp.where(above, sh, 0.0))
    sa_bits = lax.bitcast_convert_type(s_above, jnp.int32)
    rowid = lax.broadcasted_iota(jnp.int32, (8, 128), 0)
    out_ref[...] = jnp.where(rowid == 0, bstar,
                             jnp.where(rowid == 1, n_gt1, sa_bits))


def _phase_c(keys_hbm, sel_hbm, zeros_hbm, hist2_hbm, k_v, sel_v, hist2_v,
             *, per_w):
    c = lax.axis_index("c")
    s = lax.axis_index("s")
    wid = s * NC + c
    pltpu.sync_copy(zeros_hbm.at[pl.ds(0, HB2)], hist2_v)
    pltpu.sync_copy(sel_hbm.at[pl.ds(0, L)], sel_v)
    bs_vec = sel_v[...]
    ones = jnp.ones((L,), jnp.int32)

    for ch in range(per_w // CH):
        base = wid * per_w + ch * CH
        pltpu.sync_copy(keys_hbm.at[pl.ds(base, CH)], k_v)

        def body(i):
            key = k_v[pl.ds(i * L, L)]
            hi = lax.shift_right_logical(key, 16)
            low = jnp.bitwise_and(key, jnp.int32(0xFFFF))
            plsc.addupdate_scatter(hist2_v, [low], ones, mask=hi == bs_vec)

        plsc.parallel_loop(0, CH // L, 1, unroll=8)(body)

    pltpu.sync_copy(hist2_v, hist2_hbm.at[wid])


def _phase_d(hist2_ref, sel_ref, out_ref, *, k):
    h2 = jnp.sum(hist2_ref[...], axis=0)
    j = (lax.broadcasted_iota(jnp.int32, h2.shape, 0) * h2.shape[1]
         + lax.broadcasted_iota(jnp.int32, h2.shape, 1))
    bstar = sel_ref[0, 0]
    n_gt1 = sel_ref[1, 0]
    s_above = lax.bitcast_convert_type(sel_ref[2, 0], jnp.float32)
    r1 = k - n_gt1

    def step(i, prefix):
        trial = prefix + (jnp.int32(1) << (jnp.int32(15) - i))
        cnt = jnp.sum(jnp.where(j >= trial, h2, 0))
        return jnp.where(cnt >= r1, trial, prefix)

    low = lax.fori_loop(0, 16, step, jnp.int32(0))
    vals = lax.bitcast_convert_type((bstar << 16) + j, jnp.float32)
    above = j > low
    n2 = jnp.sum(jnp.where(above, h2, 0))
    s2 = jnp.sum(jnp.where(above, h2.astype(jnp.float32) * vals, 0.0))
    t_val = lax.bitcast_convert_type((bstar << 16) + low, jnp.float32)
    rem = (r1 - n2).astype(jnp.float32)
    out_ref[0, 0] = (s_above + s2 + rem * t_val) / jnp.float32(k)


def kernel(inputs, targets):
    n_total = inputs.size
    k = int(0.6 * n_total)
    per_w = n_total // NW
    x = inputs.reshape(n_total)
    y = targets.reshape(n_total)
    zeros = jnp.zeros((HB2,), jnp.int32)
    zerosf = jnp.zeros((HB1,), jnp.float32)

    mesh = plsc.VectorSubcoreMesh(core_axis_name="c", subcore_axis_name="s")
    sc_params = pltpu.CompilerParams(needs_layout_passes=False)

    keys, hist1, shist1 = pl.kernel(
        functools.partial(_phase_a, per_w=per_w),
        mesh=mesh,
        compiler_params=sc_params,
        out_type=[jax.ShapeDtypeStruct((n_total,), jnp.int32),
                  jax.ShapeDtypeStruct((NW, HB1), jnp.int32),
                  jax.ShapeDtypeStruct((NW, HB1), jnp.float32)],
        scratch_types=[pltpu.VMEM((CH,), jnp.float32),
                       pltpu.VMEM((CH,), jnp.float32),
                       pltpu.VMEM((CH,), jnp.int32),
                       pltpu.VMEM((HB1,), jnp.int32),
                       pltpu.VMEM((HB1,), jnp.float32)],
    )(x, y, zeros, zerosf)

    sel = pl.pallas_call(
        functools.partial(_phase_b, k=k),
        out_shape=jax.ShapeDtypeStruct((8, 128), jnp.int32),
        in_specs=[pl.BlockSpec(memory_space=pltpu.VMEM),
                  pl.BlockSpec(memory_space=pltpu.VMEM)],
        out_specs=pl.BlockSpec(memory_space=pltpu.VMEM),
    )(hist1.reshape(NW, HB1 // 128, 128), shist1.reshape(NW, HB1 // 128, 128))

    hist2 = pl.kernel(
        functools.partial(_phase_c, per_w=per_w),
        mesh=mesh,
        compiler_params=sc_params,
        out_type=jax.ShapeDtypeStruct((NW, HB2), jnp.int32),
        scratch_types=[pltpu.VMEM((CH,), jnp.int32),
                       pltpu.VMEM((L,), jnp.int32),
                       pltpu.VMEM((HB2,), jnp.int32)],
    )(keys, sel.reshape(HB1 // 32), zeros)

    out = pl.pallas_call(
        functools.partial(_phase_d, k=k),
        out_shape=jax.ShapeDtypeStruct((1, 1), jnp.float32),
        in_specs=[pl.BlockSpec(memory_space=pltpu.VMEM),
                  pl.BlockSpec(memory_space=pltpu.VMEM)],
        out_specs=pl.BlockSpec(memory_space=pltpu.SMEM),
    )(hist2.reshape(NW, HB2 // 128, 128), sel)

    return out[0, 0]


# speedup vs baseline: 1.4984x; 1.1090x over previous
import functools

import jax
import jax.numpy as jnp
from jax import lax
from jax.experimental import pallas as pl
from jax.experimental.pallas import tpu as pltpu
from jax.experimental.pallas import tpu_sc as plsc

NC = 2
NS = 16
NW = NC * NS
L = 16

HB1 = 32768
HB2 = 65536
CH = 8192


def _phase_a(x_hbm, y_hbm, zeros_hbm, zerosf_hbm, keys_hbm, hist_hbm,
             shist_hbm, x_v, y_v, k_v, hist_v, shist_v, sem_x, sem_y, sem_o,
             *, per_w):
    c = lax.axis_index("c")
    s = lax.axis_index("s")
    wid = s * NC + c
    n_ch = per_w // CH
    pltpu.sync_copy(zeros_hbm.at[pl.ds(0, HB1)], hist_v)
    pltpu.sync_copy(zerosf_hbm.at[pl.ds(0, HB1)], shist_v)
    ones = jnp.ones((L,), jnp.int32)

    def start_in(ch):
        b = ch % 2
        base = wid * per_w + ch * CH
        hx = pltpu.async_copy(x_hbm.at[pl.ds(base, CH)], x_v.at[b], sem_x.at[b])
        hy = pltpu.async_copy(y_hbm.at[pl.ds(base, CH)], y_v.at[b], sem_y.at[b])
        return hx, hy

    pending = start_in(0)
    out_pending = [None, None]
    for ch in range(n_ch):
        b = ch % 2
        hx, hy = pending
        hx.wait()
        hy.wait()
        if ch + 1 < n_ch:
            pending = start_in(ch + 1)
        if out_pending[b] is not None:
            out_pending[b].wait()

        def body(i):
            sl = pl.ds(i * L, L)
            d = x_v[b, sl] - y_v[b, sl]
            a = jnp.abs(d)
            loss = jnp.where(a < 1.0, 0.5 * d * d, a - 0.5)
            key = plsc.bitcast(loss, jnp.int32)
            k_v[b, sl] = key
            idx = lax.shift_right_logical(key, 16)
            plsc.addupdate_scatter(hist_v, [idx], ones)
            plsc.addupdate_scatter(shist_v, [idx], loss)

        plsc.parallel_loop(0, CH // L, 1, unroll=8)(body)
        out_pending[b] = pltpu.async_copy(
            k_v.at[b], keys_hbm.at[pl.ds(wid * per_w + ch * CH, CH)],
            sem_o.at[b])

    for h in out_pending:
        if h is not None:
            h.wait()
    pltpu.sync_copy(hist_v, hist_hbm.at[wid])
    pltpu.sync_copy(shist_v, shist_hbm.at[wid])


def _phase_b(hist_ref, shist_ref, out_ref, *, k):
    h = jnp.sum(hist_ref[...], axis=0)
    sh = jnp.sum(shist_ref[...], axis=0)
    idx = (lax.broadcasted_iota(jnp.int32, h.shape, 0) * h.shape[1]
           + lax.broadcasted_iota(jnp.int32, h.shape, 1))

    def step(i, prefix):
        trial = prefix + (jnp.int32(1) << (jnp.int32(14) - i))
        cnt = jnp.sum(jnp.where(idx >= trial, h, 0))
        return jnp.where(cnt >= k, trial, prefix)

    bstar = lax.fori_loop(0, 15, step, jnp.int32(0))
    above = idx > bstar
    n_gt1 = jnp.sum(jnp.where(above, h, 0))
    s_above = jnp.sum(jnp.where(above, sh, 0.0))
    sa_bits = lax.bitcast_convert_type(s_above, jnp.int32)
    rowid = lax.broadcasted_iota(jnp.int32, (8, 128), 0)
    out_ref[...] = jnp.where(rowid == 0, bstar,
                             jnp.where(rowid == 1, n_gt1, sa_bits))


def _phase_c(keys_hbm, sel_hbm, zeros_hbm, hist2_hbm, k_v, sel_v, hist2_v,
             sem_k, *, per_w):
    c = lax.axis_index("c")
    s = lax.axis_index("s")
    wid = s * NC + c
    n_ch = per_w // CH
    pltpu.sync_copy(zeros_hbm.at[pl.ds(0, HB2)], hist2_v)
    pltpu.sync_copy(sel_hbm.at[pl.ds(0, L)], sel_v)
    bs_vec = sel_v[...]
    ones = jnp.ones((L,), jnp.int32)

    def start_in(ch):
        b = ch % 2
        base = wid * per_w + ch * CH
        return pltpu.async_copy(keys_hbm.at[pl.ds(base, CH)], k_v.at[b],
                                sem_k.at[b])

    pending = start_in(0)
    for ch in range(n_ch):
        b = ch % 2
        pending.wait()
        if ch + 1 < n_ch:
            pending = start_in(ch + 1)

        def body(i):
            key = k_v[b, pl.ds(i * L, L)]
            hi = lax.shift_right_logical(key, 16)
            low = jnp.bitwise_and(key, jnp.int32(0xFFFF))
            plsc.addupdate_scatter(hist2_v, [low], ones, mask=hi == bs_vec)

        plsc.parallel_loop(0, CH // L, 1, unroll=8)(body)

    pltpu.sync_copy(hist2_v, hist2_hbm.at[wid])


def _phase_d(hist2_ref, sel_ref, out_ref, *, k):
    h2 = jnp.sum(hist2_ref[...], axis=0)
    j = (lax.broadcasted_iota(jnp.int32, h2.shape, 0) * h2.shape[1]
         + lax.broadcasted_iota(jnp.int32, h2.shape, 1))
    bstar = sel_ref[0, 0]
    n_gt1 = sel_ref[1, 0]
    s_above = lax.bitcast_convert_type(sel_ref[2, 0], jnp.float32)
    r1 = k - n_gt1

    def step(i, prefix):
        trial = prefix + (jnp.int32(1) << (jnp.int32(15) - i))
        cnt = jnp.sum(jnp.where(j >= trial, h2, 0))
        return jnp.where(cnt >= r1, trial, prefix)

    low = lax.fori_loop(0, 16, step, jnp.int32(0))
    vals = lax.bitcast_convert_type((bstar << 16) + j, jnp.float32)
    above = j > low
    n2 = jnp.sum(jnp.where(above, h2, 0))
    s2 = jnp.sum(jnp.where(above, h2.astype(jnp.float32) * vals, 0.0))
    t_val = lax.bitcast_convert_type((bstar << 16) + low, jnp.float32)
    rem = (r1 - n2).astype(jnp.float32)
    out_ref[0, 0] = (s_above + s2 + rem * t_val) / jnp.float32(k)


def kernel(inputs, targets):
    n_total = inputs.size
    k = int(0.6 * n_total)
    per_w = n_total // NW
    x = inputs.reshape(n_total)
    y = targets.reshape(n_total)
    zeros = jnp.zeros((HB2,), jnp.int32)
    zerosf = jnp.zeros((HB1,), jnp.float32)

    mesh = plsc.VectorSubcoreMesh(core_axis_name="c", subcore_axis_name="s")
    sc_params = pltpu.CompilerParams(needs_layout_passes=False)

    keys, hist1, shist1 = pl.kernel(
        functools.partial(_phase_a, per_w=per_w),
        mesh=mesh,
        compiler_params=sc_params,
        out_type=[jax.ShapeDtypeStruct((n_total,), jnp.int32),
                  jax.ShapeDtypeStruct((NW, HB1), jnp.int32),
                  jax.ShapeDtypeStruct((NW, HB1), jnp.float32)],
        scratch_types=[pltpu.VMEM((2, CH), jnp.float32),
                       pltpu.VMEM((2, CH), jnp.float32),
                       pltpu.VMEM((2, CH), jnp.int32),
                       pltpu.VMEM((HB1,), jnp.int32),
                       pltpu.VMEM((HB1,), jnp.float32),
                       pltpu.SemaphoreType.DMA((2,)),
                       pltpu.SemaphoreType.DMA((2,)),
                       pltpu.SemaphoreType.DMA((2,))],
    )(x, y, zeros, zerosf)

    sel = pl.pallas_call(
        functools.partial(_phase_b, k=k),
        out_shape=jax.ShapeDtypeStruct((8, 128), jnp.int32),
        in_specs=[pl.BlockSpec(memory_space=pltpu.VMEM),
                  pl.BlockSpec(memory_space=pltpu.VMEM)],
        out_specs=pl.BlockSpec(memory_space=pltpu.VMEM),
    )(hist1.reshape(NW, HB1 // 128, 128), shist1.reshape(NW, HB1 // 128, 128))

    hist2 = pl.kernel(
        functools.partial(_phase_c, per_w=per_w),
        mesh=mesh,
        compiler_params=sc_params,
        out_type=jax.ShapeDtypeStruct((NW, HB2), jnp.int32),
        scratch_types=[pltpu.VMEM((2, CH), jnp.int32),
                       pltpu.VMEM((L,), jnp.int32),
                       pltpu.VMEM((HB2,), jnp.int32),
                       pltpu.SemaphoreType.DMA((2,))],
    )(keys, sel.reshape(HB1 // 32), zeros)

    out = pl.pallas_call(
        functools.partial(_phase_d, k=k),
        out_shape=jax.ShapeDtypeStruct((1, 1), jnp.float32),
        in_specs=[pl.BlockSpec(memory_space=pltpu.VMEM),
                  pl.BlockSpec(memory_space=pltpu.VMEM)],
        out_specs=pl.BlockSpec(memory_space=pltpu.SMEM),
    )(hist2.reshape(NW, HB2 // 128, 128), sel)

    return out[0, 0]


# speedup vs baseline: 2.4143x; 1.6112x over previous
import functools

import jax
import jax.numpy as jnp
from jax import lax
from jax.experimental import pallas as pl
from jax.experimental.pallas import tpu as pltpu
from jax.experimental.pallas import tpu_sc as plsc

NC = 2
NS = 16
NW = NC * NS
L = 16

HB1 = 32768
HB2 = 65536
G1 = HB1 // 1024
G2 = HB2 // 1024
CH = 8192


def _split3(idx):
    return (lax.shift_right_logical(idx, 10),
            jnp.bitwise_and(lax.shift_right_logical(idx, 7), jnp.int32(7)),
            jnp.bitwise_and(idx, jnp.int32(127)))


def _phase_a(x_hbm, y_hbm, zeros_hbm, zerosf_hbm, keys_hbm, hist_hbm,
             shist_hbm, x_v, y_v, k_v, hist_v, shist_v, sem_x, sem_y, sem_o,
             *, rows_w, n_cols):
    c = lax.axis_index("c")
    s = lax.axis_index("s")
    wid = s * NC + c
    rch = CH // n_cols
    n_ch = rows_w // rch
    pltpu.sync_copy(zeros_hbm.at[pl.ds(0, G1)], hist_v)
    pltpu.sync_copy(zerosf_hbm.at[pl.ds(0, G1)], shist_v)
    ones = jnp.ones((L,), jnp.int32)
    lg = n_cols // L

    def start_in(ch):
        b = ch % 2
        r0 = (wid * rows_w + ch * rch)
        hx = pltpu.async_copy(x_hbm.at[pl.ds(r0, rch), :], x_v.at[b],
                              sem_x.at[b])
        hy = pltpu.async_copy(y_hbm.at[pl.ds(r0, rch), :], y_v.at[b],
                              sem_y.at[b])
        return hx, hy

    pending = start_in(0)
    out_pending = [None, None]
    for ch in range(n_ch):
        b = ch % 2
        hx, hy = pending
        hx.wait()
        hy.wait()
        if ch + 1 < n_ch:
            pending = start_in(ch + 1)
        if out_pending[b] is not None:
            out_pending[b].wait()

        def body(i):
            r = i // lg
            sl = pl.ds((i % lg) * L, L)
            d = x_v[b, r, sl] - y_v[b, r, sl]
            a = jnp.abs(d)
            loss = jnp.where(a < 1.0, 0.5 * d * d, a - 0.5)
            key = plsc.bitcast(loss, jnp.int32)
            k_v[b, r, sl] = key
            i3 = _split3(lax.shift_right_logical(key, 16))
            plsc.addupdate_scatter(hist_v, i3, ones)
            plsc.addupdate_scatter(shist_v, i3, loss)

        plsc.parallel_loop(0, CH // L, 1, unroll=8)(body)
        out_pending[b] = pltpu.async_copy(
            k_v.at[b], keys_hbm.at[pl.ds(wid * rows_w + ch * rch, rch), :],
            sem_o.at[b])

    for h in out_pending:
        if h is not None:
            h.wait()
    pltpu.sync_copy(hist_v, hist_hbm.at[pl.ds(wid * G1, G1)])
    pltpu.sync_copy(shist_v, shist_hbm.at[pl.ds(wid * G1, G1)])


def _phase_b(hist_ref, shist_ref, out_ref, *, k):
    h = jnp.sum(hist_ref[...], axis=0)
    sh = jnp.sum(shist_ref[...], axis=0)
    idx = (lax.broadcasted_iota(jnp.int32, h.shape, 0) * 1024
           + lax.broadcasted_iota(jnp.int32, h.shape, 1) * 128
           + lax.broadcasted_iota(jnp.int32, h.shape, 2))

    def step(i, prefix):
        trial = prefix + (jnp.int32(1) << (jnp.int32(14) - i))
        cnt = jnp.sum(jnp.where(idx >= trial, h, 0))
        return jnp.where(cnt >= k, trial, prefix)

    bstar = lax.fori_loop(0, 15, step, jnp.int32(0))
    above = idx > bstar
    n_gt1 = jnp.sum(jnp.where(above, h, 0))
    s_above = jnp.sum(jnp.where(above, sh, 0.0))
    sa_bits = lax.bitcast_convert_type(s_above, jnp.int32)
    rowid = lax.broadcasted_iota(jnp.int32, (8, 128), 0)
    out_ref[...] = jnp.where(rowid == 0, bstar,
                             jnp.where(rowid == 1, n_gt1, sa_bits))


def _phase_c(keys_hbm, sel_hbm, zeros_hbm, hist2_hbm, k_v, sel_v, hist2_v,
             sem_k, *, rows_w, n_cols):
    c = lax.axis_index("c")
    s = lax.axis_index("s")
    wid = s * NC + c
    rch = CH // n_cols
    n_ch = rows_w // rch
    pltpu.sync_copy(zeros_hbm.at[pl.ds(0, G2)], hist2_v)
    pltpu.sync_copy(sel_hbm.at[0, pl.ds(0, L)], sel_v)
    bs_vec = sel_v[...]
    ones = jnp.ones((L,), jnp.int32)
    lg = n_cols // L

    def start_in(ch):
        b = ch % 2
        r0 = wid * rows_w + ch * rch
        return pltpu.async_copy(keys_hbm.at[pl.ds(r0, rch), :], k_v.at[b],
                                sem_k.at[b])

    pending = start_in(0)
    for ch in range(n_ch):
        b = ch % 2
        pending.wait()
        if ch + 1 < n_ch:
            pending = start_in(ch + 1)

        def body(i):
            key = k_v[b, i // lg, pl.ds((i % lg) * L, L)]
            hi = lax.shift_right_logical(key, 16)
            low = jnp.bitwise_and(key, jnp.int32(0xFFFF))
            plsc.addupdate_scatter(hist2_v, _split3(low), ones,
                                   mask=hi == bs_vec)

        plsc.parallel_loop(0, CH // L, 1, unroll=8)(body)

    pltpu.sync_copy(hist2_v, hist2_hbm.at[pl.ds(wid * G2, G2)])


def _phase_d(hist2_ref, sel_ref, out_ref, *, k):
    h2 = jnp.sum(hist2_ref[...], axis=0)
    j = (lax.broadcasted_iota(jnp.int32, h2.shape, 0) * 1024
         + lax.broadcasted_iota(jnp.int32, h2.shape, 1) * 128
         + lax.broadcasted_iota(jnp.int32, h2.shape, 2))
    bstar = sel_ref[0, 0]
    n_gt1 = sel_ref[1, 0]
    s_above = lax.bitcast_convert_type(sel_ref[2, 0], jnp.float32)
    r1 = k - n_gt1

    def step(i, prefix):
        trial = prefix + (jnp.int32(1) << (jnp.int32(15) - i))
        cnt = jnp.sum(jnp.where(j >= trial, h2, 0))
        return jnp.where(cnt >= r1, trial, prefix)

    low = lax.fori_loop(0, 16, step, jnp.int32(0))
    vals = lax.bitcast_convert_type((bstar << 16) + j, jnp.float32)
    above = j > low
    n2 = jnp.sum(jnp.where(above, h2, 0))
    s2 = jnp.sum(jnp.where(above, h2.astype(jnp.float32) * vals, 0.0))
    t_val = lax.bitcast_convert_type((bstar << 16) + low, jnp.float32)
    rem = (r1 - n2).astype(jnp.float32)
    out_ref[0, 0] = (s_above + s2 + rem * t_val) / jnp.float32(k)


def kernel(inputs, targets):
    n_total = inputs.size
    k = int(0.6 * n_total)
    n_cols = inputs.shape[-1]
    n_rows = n_total // n_cols
    rows_w = n_rows // NW
    rch = CH // n_cols
    x = inputs.reshape(n_rows, n_cols)
    y = targets.reshape(n_rows, n_cols)
    zeros = jnp.zeros((G2, 8, 128), jnp.int32)
    zerosf = jnp.zeros((G1, 8, 128), jnp.float32)

    mesh = plsc.VectorSubcoreMesh(core_axis_name="c", subcore_axis_name="s")
    sc_params = pltpu.CompilerParams(needs_layout_passes=False)

    keys, hist1, shist1 = pl.kernel(
        functools.partial(_phase_a, rows_w=rows_w, n_cols=n_cols),
        mesh=mesh,
        compiler_params=sc_params,
        out_type=[jax.ShapeDtypeStruct((n_rows, n_cols), jnp.int32),
                  jax.ShapeDtypeStruct((NW * G1, 8, 128), jnp.int32),
                  jax.ShapeDtypeStruct((NW * G1, 8, 128), jnp.float32)],
        scratch_types=[pltpu.VMEM((2, rch, n_cols), jnp.float32),
                       pltpu.VMEM((2, rch, n_cols), jnp.float32),
                       pltpu.VMEM((2, rch, n_cols), jnp.int32),
                       pltpu.VMEM((G1, 8, 128), jnp.int32),
                       pltpu.VMEM((G1, 8, 128), jnp.float32),
                       pltpu.SemaphoreType.DMA((2,)),
                       pltpu.SemaphoreType.DMA((2,)),
                       pltpu.SemaphoreType.DMA((2,))],
    )(x, y, zeros, zerosf)

    sel = pl.pallas_call(
        functools.partial(_phase_b, k=k),
        out_shape=jax.ShapeDtypeStruct((8, 128), jnp.int32),
        in_specs=[pl.BlockSpec(memory_space=pltpu.VMEM),
                  pl.BlockSpec(memory_space=pltpu.VMEM)],
        out_specs=pl.BlockSpec(memory_space=pltpu.VMEM),
    )(hist1.reshape(NW, G1, 8, 128), shist1.reshape(NW, G1, 8, 128))

    hist2 = pl.kernel(
        functools.partial(_phase_c, rows_w=rows_w, n_cols=n_cols),
        mesh=mesh,
        compiler_params=sc_params,
        out_type=jax.ShapeDtypeStruct((NW * G2, 8, 128), jnp.int32),
        scratch_types=[pltpu.VMEM((2, rch, n_cols), jnp.int32),
                       pltpu.VMEM((L,), jnp.int32),
                       pltpu.VMEM((G2, 8, 128), jnp.int32),
                       pltpu.SemaphoreType.DMA((2,))],
    )(keys, sel, zeros)

    out = pl.pallas_call(
        functools.partial(_phase_d, k=k),
        out_shape=jax.ShapeDtypeStruct((1, 1), jnp.float32),
        in_specs=[pl.BlockSpec(memory_space=pltpu.VMEM),
                  pl.BlockSpec(memory_space=pltpu.VMEM)],
        out_specs=pl.BlockSpec(memory_space=pltpu.SMEM),
    )(hist2.reshape(NW, G2, 8, 128), sel)

    return out[0, 0]


# speedup vs baseline: 3.2369x; 1.3407x over previous
import functools

import jax
import jax.numpy as jnp
from jax import lax
from jax.experimental import pallas as pl
from jax.experimental.pallas import tpu as pltpu
from jax.experimental.pallas import tpu_sc as plsc

NC = 2
NS = 16
NW = NC * NS
L = 16

HB1 = 32768
HB2 = 8192
G1 = HB1 // 1024
G2 = HB2 // 1024
CH = 8192


def _split3(idx):
    return (lax.shift_right_logical(idx, 10),
            jnp.bitwise_and(lax.shift_right_logical(idx, 7), jnp.int32(7)),
            jnp.bitwise_and(idx, jnp.int32(127)))


def _zero3(ref, nblk, dtype):
    z = jnp.zeros((L,), dtype)

    def body(i):
        ref[lax.shift_right_logical(i, 6),
            jnp.bitwise_and(lax.shift_right_logical(i, 3), jnp.int32(7)),
            pl.ds(jnp.bitwise_and(i, jnp.int32(7)) * L, L)] = z

    plsc.parallel_loop(0, nblk * 64, 1, unroll=8)(body)


def _phase_a(x_hbm, y_hbm, keys_hbm, hist_hbm, shist_hbm, x_v, y_v, k_v,
             hist_v, shist_v, sem_x, sem_y, sem_o, *, rows_w, n_cols):
    c = lax.axis_index("c")
    s = lax.axis_index("s")
    wid = s * NC + c
    rch = CH // n_cols
    n_ch = rows_w // rch
    ones = jnp.ones((L,), jnp.int32)
    lg = n_cols // L

    def start_in(ch):
        b = ch % 2
        r0 = (wid * rows_w + ch * rch)
        hx = pltpu.async_copy(x_hbm.at[pl.ds(r0, rch), :], x_v.at[b],
                              sem_x.at[b])
        hy = pltpu.async_copy(y_hbm.at[pl.ds(r0, rch), :], y_v.at[b],
                              sem_y.at[b])
        return hx, hy

    pending = start_in(0)
    _zero3(hist_v, G1, jnp.int32)
    _zero3(shist_v, G1, jnp.float32)
    out_pending = [None, None]
    for ch in range(n_ch):
        b = ch % 2
        hx, hy = pending
        hx.wait()
        hy.wait()
        if ch + 1 < n_ch:
            pending = start_in(ch + 1)
        if out_pending[b] is not None:
            out_pending[b].wait()

        def body(i):
            r = i // lg
            sl = pl.ds((i % lg) * L, L)
            d = x_v[b, r, sl] - y_v[b, r, sl]
            a = jnp.abs(d)
            loss = jnp.where(a < 1.0, 0.5 * d * d, a - 0.5)
            key = plsc.bitcast(loss, jnp.int32)
            k_v[b, r, sl] = key
            i3 = _split3(lax.shift_right_logical(key, 16))
            plsc.addupdate_scatter(hist_v, i3, ones)
            plsc.addupdate_scatter(shist_v, i3, loss)

        plsc.parallel_loop(0, CH // L, 1, unroll=8)(body)
        out_pending[b] = pltpu.async_copy(
            k_v.at[b], keys_hbm.at[pl.ds(wid * rows_w + ch * rch, rch), :],
            sem_o.at[b])

    for h in out_pending:
        if h is not None:
            h.wait()
    pltpu.sync_copy(hist_v, hist_hbm.at[pl.ds(wid * G1, G1)])
    pltpu.sync_copy(shist_v, shist_hbm.at[pl.ds(wid * G1, G1)])


def _phase_b(hist_ref, shist_ref, out_ref, *, k):
    h = jnp.sum(hist_ref[...], axis=0)
    sh = jnp.sum(shist_ref[...], axis=0)
    idx = (lax.broadcasted_iota(jnp.int32, h.shape, 0) * 1024
           + lax.broadcasted_iota(jnp.int32, h.shape, 1) * 128
           + lax.broadcasted_iota(jnp.int32, h.shape, 2))

    def step(i, prefix):
        trial = prefix + (jnp.int32(1) << (jnp.int32(14) - i))
        cnt = jnp.sum(jnp.where(idx >= trial, h, 0))
        return jnp.where(cnt >= k, trial, prefix)

    bstar = lax.fori_loop(0, 15, step, jnp.int32(0))
    above = idx > bstar
    n_gt1 = jnp.sum(jnp.where(above, h, 0))
    s_above = jnp.sum(jnp.where(above, sh, 0.0))
    sa_bits = lax.bitcast_convert_type(s_above, jnp.int32)
    rowid = lax.broadcasted_iota(jnp.int32, (8, 128), 0)
    out_ref[...] = jnp.where(rowid == 0, bstar,
                             jnp.where(rowid == 1, n_gt1, sa_bits))


def _phase_c(keys_hbm, sel_hbm, hist2_hbm, k_v, sel_v, hist2_v, sem_k,
             *, rows_w, n_cols):
    c = lax.axis_index("c")
    s = lax.axis_index("s")
    wid = s * NC + c
    rch = CH // n_cols
    n_ch = rows_w // rch
    ones = jnp.ones((L,), jnp.int32)
    lg = n_cols // L

    def start_in(ch):
        b = ch % 2
        r0 = wid * rows_w + ch * rch
        return pltpu.async_copy(keys_hbm.at[pl.ds(r0, rch), :], k_v.at[b],
                                sem_k.at[b])

    pending = start_in(0)
    _zero3(hist2_v, G2, jnp.int32)
    pltpu.sync_copy(sel_hbm.at[0, pl.ds(0, L)], sel_v)
    bs_vec = sel_v[...]
    for ch in range(n_ch):
        b = ch % 2
        pending.wait()
        if ch + 1 < n_ch:
            pending = start_in(ch + 1)

        def body(i):
            key = k_v[b, i // lg, pl.ds((i % lg) * L, L)]
            hi = lax.shift_right_logical(key, 16)
            lo13 = jnp.bitwise_and(lax.shift_right_logical(key, 3),
                                   jnp.int32(HB2 - 1))
            plsc.addupdate_scatter(hist2_v, _split3(lo13), ones,
                                   mask=hi == bs_vec)

        plsc.parallel_loop(0, CH // L, 1, unroll=8)(body)

    pltpu.sync_copy(hist2_v, hist2_hbm.at[pl.ds(wid * G2, G2)])


def _phase_d(hist2_ref, sel_ref, out_ref, *, k):
    h2 = jnp.sum(hist2_ref[...], axis=0)
    j = (lax.broadcasted_iota(jnp.int32, h2.shape, 0) * 1024
         + lax.broadcasted_iota(jnp.int32, h2.shape, 1) * 128
         + lax.broadcasted_iota(jnp.int32, h2.shape, 2))
    bstar = sel_ref[0, 0]
    n_gt1 = sel_ref[1, 0]
    s_above = lax.bitcast_convert_type(sel_ref[2, 0], jnp.float32)
    r1 = k - n_gt1

    def step(i, prefix):
        trial = prefix + (jnp.int32(1) << (jnp.int32(12) - i))
        cnt = jnp.sum(jnp.where(j >= trial, h2, 0))
        return jnp.where(cnt >= r1, trial, prefix)

    low = lax.fori_loop(0, 13, step, jnp.int32(0))
    vals = lax.bitcast_convert_type((bstar << 16) + (j << 3), jnp.float32)
    above = j > low
    n2 = jnp.sum(jnp.where(above, h2, 0))
    s2 = jnp.sum(jnp.where(above, h2.astype(jnp.float32) * vals, 0.0))
    t_val = lax.bitcast_convert_type((bstar << 16) + (low << 3), jnp.float32)
    rem = (r1 - n2).astype(jnp.float32)
    out_ref[0, 0] = (s_above + s2 + rem * t_val) / jnp.float32(k)


def kernel(inputs, targets):
    n_total = inputs.size
    k = int(0.6 * n_total)
    n_cols = inputs.shape[-1]
    n_rows = n_total // n_cols
    rows_w = n_rows // NW
    rch = CH // n_cols
    x = inputs.reshape(n_rows, n_cols)
    y = targets.reshape(n_rows, n_cols)

    mesh = plsc.VectorSubcoreMesh(core_axis_name="c", subcore_axis_name="s")
    sc_params = pltpu.CompilerParams(needs_layout_passes=False)

    keys, hist1, shist1 = pl.kernel(
        functools.partial(_phase_a, rows_w=rows_w, n_cols=n_cols),
        mesh=mesh,
        compiler_params=sc_params,
        out_type=[jax.ShapeDtypeStruct((n_rows, n_cols), jnp.int32),
                  jax.ShapeDtypeStruct((NW * G1, 8, 128), jnp.int32),
                  jax.ShapeDtypeStruct((NW * G1, 8, 128), jnp.float32)],
        scratch_types=[pltpu.VMEM((2, rch, n_cols), jnp.float32),
                       pltpu.VMEM((2, rch, n_cols), jnp.float32),
                       pltpu.VMEM((2, rch, n_cols), jnp.int32),
                       pltpu.VMEM((G1, 8, 128), jnp.int32),
                       pltpu.VMEM((G1, 8, 128), jnp.float32),
                       pltpu.SemaphoreType.DMA((2,)),
                       pltpu.SemaphoreType.DMA((2,)),
                       pltpu.SemaphoreType.DMA((2,))],
    )(x, y)

    sel = pl.pallas_call(
        functools.partial(_phase_b, k=k),
        out_shape=jax.ShapeDtypeStruct((8, 128), jnp.int32),
        in_specs=[pl.BlockSpec(memory_space=pltpu.VMEM),
                  pl.BlockSpec(memory_space=pltpu.VMEM)],
        out_specs=pl.BlockSpec(memory_space=pltpu.VMEM),
    )(hist1.reshape(NW, G1, 8, 128), shist1.reshape(NW, G1, 8, 128))

    hist2 = pl.kernel(
        functools.partial(_phase_c, rows_w=rows_w, n_cols=n_cols),
        mesh=mesh,
        compiler_params=sc_params,
        out_type=jax.ShapeDtypeStruct((NW * G2, 8, 128), jnp.int32),
        scratch_types=[pltpu.VMEM((2, rch, n_cols), jnp.int32),
                       pltpu.VMEM((L,), jnp.int32),
                       pltpu.VMEM((G2, 8, 128), jnp.int32),
                       pltpu.SemaphoreType.DMA((2,))],
    )(keys, sel)

    out = pl.pallas_call(
        functools.partial(_phase_d, k=k),
        out_shape=jax.ShapeDtypeStruct((1, 1), jnp.float32),
        in_specs=[pl.BlockSpec(memory_space=pltpu.VMEM),
                  pl.BlockSpec(memory_space=pltpu.VMEM)],
        out_specs=pl.BlockSpec(memory_space=pltpu.SMEM),
    )(hist2.reshape(NW, G2, 8, 128), sel)

    return out[0, 0]


# speedup vs baseline: 3.7525x; 1.1593x over previous
import functools

import jax
import jax.numpy as jnp
from jax import lax
from jax.experimental import pallas as pl
from jax.experimental.pallas import tpu as pltpu
from jax.experimental.pallas import tpu_sc as plsc

NC = 2
NS = 16
NW = NC * NS
L = 16

HB1 = 4096
HB2 = 8192
G1 = HB1 // 1024
G2 = HB2 // 1024
CH = 16384


def _split3(idx):
    return (lax.shift_right_logical(idx, 10),
            jnp.bitwise_and(lax.shift_right_logical(idx, 7), jnp.int32(7)),
            jnp.bitwise_and(idx, jnp.int32(127)))


def _zero3(ref, nblk, dtype):
    z = jnp.zeros((L,), dtype)

    def body(i):
        ref[lax.shift_right_logical(i, 6),
            jnp.bitwise_and(lax.shift_right_logical(i, 3), jnp.int32(7)),
            pl.ds(jnp.bitwise_and(i, jnp.int32(7)) * L, L)] = z

    plsc.parallel_loop(0, nblk * 64, 1, unroll=8)(body)


def _phase_a(x_hbm, y_hbm, keys_hbm, hist_hbm, shist_hbm, x_v, y_v, k_v,
             hist_v, shist_v, sem_x, sem_y, sem_o, *, rows_w, n_cols):
    c = lax.axis_index("c")
    s = lax.axis_index("s")
    wid = s * NC + c
    rch = CH // n_cols
    n_ch = rows_w // rch
    ones = jnp.ones((L,), jnp.int32)
    lg = n_cols // L

    def start_in(ch):
        b = ch % 2
        r0 = (wid * rows_w + ch * rch)
        hx = pltpu.async_copy(x_hbm.at[pl.ds(r0, rch), :], x_v.at[b],
                              sem_x.at[b])
        hy = pltpu.async_copy(y_hbm.at[pl.ds(r0, rch), :], y_v.at[b],
                              sem_y.at[b])
        return hx, hy

    pending = start_in(0)
    _zero3(hist_v, G1, jnp.int32)
    _zero3(shist_v, G1, jnp.float32)
    out_pending = [None, None]
    for ch in range(n_ch):
        b = ch % 2
        hx, hy = pending
        hx.wait()
        hy.wait()
        if ch + 1 < n_ch:
            pending = start_in(ch + 1)
        if out_pending[b] is not None:
            out_pending[b].wait()

        def body(i):
            r = i // lg
            sl = pl.ds((i % lg) * L, L)
            d = x_v[b, r, sl] - y_v[b, r, sl]
            a = jnp.abs(d)
            loss = jnp.where(a < 1.0, 0.5 * d * d, a - 0.5)
            key = plsc.bitcast(loss, jnp.int32)
            k_v[b, r, sl] = key
            i3 = _split3(lax.shift_right_logical(key, 19))
            plsc.addupdate_scatter(hist_v, i3, ones)
            plsc.addupdate_scatter(shist_v, i3, loss)

        plsc.parallel_loop(0, CH // L, 1, unroll=8)(body)
        out_pending[b] = pltpu.async_copy(
            k_v.at[b], keys_hbm.at[pl.ds(wid * rows_w + ch * rch, rch), :],
            sem_o.at[b])

    for h in out_pending:
        if h is not None:
            h.wait()
    pltpu.sync_copy(hist_v, hist_hbm.at[pl.ds(wid * G1, G1)])
    pltpu.sync_copy(shist_v, shist_hbm.at[pl.ds(wid * G1, G1)])


def _phase_b(hist_ref, shist_ref, out_ref, *, k):
    h = jnp.sum(hist_ref[...], axis=0)
    sh = jnp.sum(shist_ref[...], axis=0)
    idx = (lax.broadcasted_iota(jnp.int32, h.shape, 0) * 1024
           + lax.broadcasted_iota(jnp.int32, h.shape, 1) * 128
           + lax.broadcasted_iota(jnp.int32, h.shape, 2))

    def step(i, prefix):
        trial = prefix + (jnp.int32(1) << (jnp.int32(11) - i))
        cnt = jnp.sum(jnp.where(idx >= trial, h, 0))
        return jnp.where(cnt >= k, trial, prefix)

    bstar = lax.fori_loop(0, 12, step, jnp.int32(0))
    above = idx > bstar
    n_gt1 = jnp.sum(jnp.where(above, h, 0))
    s_above = jnp.sum(jnp.where(above, sh, 0.0))
    sa_bits = lax.bitcast_convert_type(s_above, jnp.int32)
    rowid = lax.broadcasted_iota(jnp.int32, (8, 128), 0)
    out_ref[...] = jnp.where(rowid == 0, bstar,
                             jnp.where(rowid == 1, n_gt1, sa_bits))


def _phase_c(keys_hbm, sel_hbm, hist2_hbm, k_v, sel_v, hist2_v, sem_k,
             *, rows_w, n_cols):
    c = lax.axis_index("c")
    s = lax.axis_index("s")
    wid = s * NC + c
    rch = CH // n_cols
    n_ch = rows_w // rch
    ones = jnp.ones((L,), jnp.int32)
    lg = n_cols // L

    def start_in(ch):
        b = ch % 2
        r0 = wid * rows_w + ch * rch
        return pltpu.async_copy(keys_hbm.at[pl.ds(r0, rch), :], k_v.at[b],
                                sem_k.at[b])

    pending = start_in(0)
    _zero3(hist2_v, G2, jnp.int32)
    pltpu.sync_copy(sel_hbm.at[0, pl.ds(0, L)], sel_v)
    bs_vec = sel_v[...]
    for ch in range(n_ch):
        b = ch % 2
        pending.wait()
        if ch + 1 < n_ch:
            pending = start_in(ch + 1)

        def body(i):
            key = k_v[b, i // lg, pl.ds((i % lg) * L, L)]
            hi = lax.shift_right_logical(key, 19)
            lo13 = jnp.bitwise_and(lax.shift_right_logical(key, 6),
                                   jnp.int32(HB2 - 1))
            plsc.addupdate_scatter(hist2_v, _split3(lo13), ones,
                                   mask=hi == bs_vec)

        plsc.parallel_loop(0, CH // L, 1, unroll=8)(body)

    pltpu.sync_copy(hist2_v, hist2_hbm.at[pl.ds(wid * G2, G2)])


def _phase_d(hist2_ref, sel_ref, out_ref, *, k):
    h2 = jnp.sum(hist2_ref[...], axis=0)
    j = (lax.broadcasted_iota(jnp.int32, h2.shape, 0) * 1024
         + lax.broadcasted_iota(jnp.int32, h2.shape, 1) * 128
         + lax.broadcasted_iota(jnp.int32, h2.shape, 2))
    bstar = sel_ref[0, 0]
    n_gt1 = sel_ref[1, 0]
    s_above = lax.bitcast_convert_type(sel_ref[2, 0], jnp.float32)
    r1 = k - n_gt1

    def step(i, prefix):
        trial = prefix + (jnp.int32(1) << (jnp.int32(12) - i))
        cnt = jnp.sum(jnp.where(j >= trial, h2, 0))
        return jnp.where(cnt >= r1, trial, prefix)

    low = lax.fori_loop(0, 13, step, jnp.int32(0))
    vals = lax.bitcast_convert_type((bstar << 19) + (j << 6), jnp.float32)
    above = j > low
    n2 = jnp.sum(jnp.where(above, h2, 0))
    s2 = jnp.sum(jnp.where(above, h2.astype(jnp.float32) * vals, 0.0))
    t_val = lax.bitcast_convert_type((bstar << 19) + (low << 6), jnp.float32)
    rem = (r1 - n2).astype(jnp.float32)
    out_ref[0, 0] = (s_above + s2 + rem * t_val) / jnp.float32(k)


def kernel(inputs, targets):
    n_total = inputs.size
    k = int(0.6 * n_total)
    n_cols = inputs.shape[-1]
    n_rows = n_total // n_cols
    rows_w = n_rows // NW
    rch = CH // n_cols
    x = inputs.reshape(n_rows, n_cols)
    y = targets.reshape(n_rows, n_cols)

    mesh = plsc.VectorSubcoreMesh(core_axis_name="c", subcore_axis_name="s")
    sc_params = pltpu.CompilerParams(needs_layout_passes=False)

    keys, hist1, shist1 = pl.kernel(
        functools.partial(_phase_a, rows_w=rows_w, n_cols=n_cols),
        mesh=mesh,
        compiler_params=sc_params,
        out_type=[jax.ShapeDtypeStruct((n_rows, n_cols), jnp.int32),
                  jax.ShapeDtypeStruct((NW * G1, 8, 128), jnp.int32),
                  jax.ShapeDtypeStruct((NW * G1, 8, 128), jnp.float32)],
        scratch_types=[pltpu.VMEM((2, rch, n_cols), jnp.float32),
                       pltpu.VMEM((2, rch, n_cols), jnp.float32),
                       pltpu.VMEM((2, rch, n_cols), jnp.int32),
                       pltpu.VMEM((G1, 8, 128), jnp.int32),
                       pltpu.VMEM((G1, 8, 128), jnp.float32),
                       pltpu.SemaphoreType.DMA((2,)),
                       pltpu.SemaphoreType.DMA((2,)),
                       pltpu.SemaphoreType.DMA((2,))],
    )(x, y)

    sel = pl.pallas_call(
        functools.partial(_phase_b, k=k),
        out_shape=jax.ShapeDtypeStruct((8, 128), jnp.int32),
        in_specs=[pl.BlockSpec(memory_space=pltpu.VMEM),
                  pl.BlockSpec(memory_space=pltpu.VMEM)],
        out_specs=pl.BlockSpec(memory_space=pltpu.VMEM),
    )(hist1.reshape(NW, G1, 8, 128), shist1.reshape(NW, G1, 8, 128))

    hist2 = pl.kernel(
        functools.partial(_phase_c, rows_w=rows_w, n_cols=n_cols),
        mesh=mesh,
        compiler_params=sc_params,
        out_type=jax.ShapeDtypeStruct((NW * G2, 8, 128), jnp.int32),
        scratch_types=[pltpu.VMEM((2, rch, n_cols), jnp.int32),
                       pltpu.VMEM((L,), jnp.int32),
                       pltpu.VMEM((G2, 8, 128), jnp.int32),
                       pltpu.SemaphoreType.DMA((2,))],
    )(keys, sel)

    out = pl.pallas_call(
        functools.partial(_phase_d, k=k),
        out_shape=jax.ShapeDtypeStruct((1, 1), jnp.float32),
        in_specs=[pl.BlockSpec(memory_space=pltpu.VMEM),
                  pl.BlockSpec(memory_space=pltpu.VMEM)],
        out_specs=pl.BlockSpec(memory_space=pltpu.SMEM),
    )(hist2.reshape(NW, G2, 8, 128), sel)

    return out[0, 0]
